# baseline (speedup 1.0000x reference)
import functools

import jax
import jax.numpy as jnp
import numpy as np
from jax import lax
from jax.experimental import pallas as pl
from jax.experimental.pallas import tpu as pltpu
from jax.experimental.pallas import tpu_sc as plsc

N = 10000
E = 160000
D = 256
H = 256
EPS = 1e-5

NC = 2
NS = 16
LANES = 16
HALF = H // 2

CHUNK = 80
EPT = 10240
EPAD = NS * EPT
NCHUNK = EPT // CHUNK
SBC = 16
SB = NCHUNK // SBC
NPAD = 10240
ROWS_PER_TILE = NPAD // NS

MM_BLK = 400


def _mm_split_body(x_ref, w_ref, o_ref):
    h = jnp.dot(x_ref[...], w_ref[...], preferred_element_type=jnp.float32)
    o_ref[0] = h[:, :HALF]
    o_ref[1] = h[:, HALF:]


def _mm_split(x, w):
    return pl.pallas_call(
        _mm_split_body,
        grid=(N // MM_BLK,),
        in_specs=[
            pl.BlockSpec((MM_BLK, D), lambda i: (i, 0)),
            pl.BlockSpec((D, H), lambda i: (0, 0)),
        ],
        out_specs=pl.BlockSpec((2, MM_BLK, HALF), lambda i: (0, i, 0)),
        out_shape=jax.ShapeDtypeStruct((2, N, HALF), jnp.float32),
    )(x, w)


def _mm_bn_body(a_ref, w_ref, s_ref, b_ref, o_ref):
    h0 = jnp.maximum(a_ref[0] * s_ref[0] + b_ref[0], 0.0)
    h1 = jnp.maximum(a_ref[1] * s_ref[1] + b_ref[1], 0.0)
    out = jnp.dot(h0, w_ref[:HALF, :], preferred_element_type=jnp.float32)
    out += jnp.dot(h1, w_ref[HALF:, :], preferred_element_type=jnp.float32)
    o_ref[0] = out[:, :HALF]
    o_ref[1] = out[:, HALF:]


def _mm_bn_split(a, w, scale, beta):
    return pl.pallas_call(
        _mm_bn_body,
        grid=(N // MM_BLK,),
        in_specs=[
            pl.BlockSpec((2, MM_BLK, HALF), lambda i: (0, i, 0)),
            pl.BlockSpec((H, H), lambda i: (0, 0)),
            pl.BlockSpec((2, 1, HALF), lambda i: (0, 0, 0)),
            pl.BlockSpec((2, 1, HALF), lambda i: (0, 0, 0)),
        ],
        out_specs=pl.BlockSpec((2, MM_BLK, HALF), lambda i: (0, i, 0)),
        out_shape=jax.ShapeDtypeStruct((2, N, HALF), jnp.float32),
    )(a, w, scale, beta)


def _sc_agg_body(hs_hbm, src_hbm, dst_hbm, w_hbm, zeros_hbm, out_hbm,
                 srcS0, srcS1, dst_sb, w_sb, rowsA, rowsB, acc_sh,
                 gsA, gsB, ssA, ssB):
    c = lax.axis_index("c")
    s = lax.axis_index("s")

    pltpu.sync_copy(zeros_hbm, acc_sh.at[pl.ds(s * ROWS_PER_TILE, ROWS_PER_TILE)])
    plsc.subcore_barrier()

    def mul_rows(rows, local):
        def grp_body(g, _):
            wvec = w_sb[local, pl.ds(g * LANES, LANES)]
            for i16 in range(LANES):
                wsplat = jnp.full((LANES,), wvec[i16], dtype=jnp.float32)
                i = g * LANES + i16
                for j in range(HALF // LANES):
                    sl = pl.ds(j * LANES, LANES)
                    rows[i, sl] = rows[i, sl] * wsplat
            return ()
        lax.fori_loop(0, CHUNK // LANES, grp_body, ())

    def g_start(src_ref, local, rows, sem):
        pltpu.async_copy(hs_hbm.at[src_ref.at[local]], rows, sem)

    def g_wait(src_ref, rows, sem):
        pltpu.make_async_copy(hs_hbm.at[src_ref.at[0]], rows, sem).wait()

    def s_start(rows, local, sem):
        return pltpu.async_copy(rows, acc_sh.at[dst_sb.at[local]], sem, add=True)

    srcS = (srcS0, srcS1)
    for sb in range(SB):
        cur = srcS[sb % 2]
        nxt = srcS[(sb + 1) % 2]
        if sb == 0:
            pltpu.sync_copy(src_hbm.at[c, s, 0], cur)
        pltpu.sync_copy(dst_hbm.at[s, sb], dst_sb)
        pltpu.sync_copy(w_hbm.at[s, sb], w_sb)
        if sb + 1 < SB:
            pltpu.sync_copy(src_hbm.at[c, s, sb + 1], nxt)
        if sb == 0:
            g_start(cur, 0, rowsA, gsA)
            g_start(cur, 1, rowsB, gsB)

        def pair(m, _):
            a = 2 * m
            b = a + 1
            g_wait(cur, rowsA, gsA)
            mul_rows(rowsA, a)
            dA = s_start(rowsA, a, ssA)
            g_wait(cur, rowsB, gsB)
            mul_rows(rowsB, b)
            dB = s_start(rowsB, b, ssB)
            dA.wait()
            g_start(cur, a + 2, rowsA, gsA)
            dB.wait()
            g_start(cur, b + 2, rowsB, gsB)
            return ()

        lax.fori_loop(0, SBC // 2 - 1, pair, ())

        g_wait(cur, rowsA, gsA)
        mul_rows(rowsA, SBC - 2)
        dA = s_start(rowsA, SBC - 2, ssA)
        g_wait(cur, rowsB, gsB)
        mul_rows(rowsB, SBC - 1)
        dB = s_start(rowsB, SBC - 1, ssB)
        dA.wait()
        dB.wait()
        if sb + 1 < SB:
            g_start(nxt, 0, rowsA, gsA)
            g_start(nxt, 1, rowsB, gsB)

    plsc.subcore_barrier()

    r0 = s * ROWS_PER_TILE
    pltpu.sync_copy(acc_sh.at[pl.ds(r0, ROWS_PER_TILE)],
                    out_hbm.at[pl.ds(c * NPAD + r0, ROWS_PER_TILE)])


@functools.partial(
    pl.kernel,
    out_type=jax.ShapeDtypeStruct((2 * NPAD, HALF), jnp.float32),
    mesh=plsc.VectorSubcoreMesh(core_axis_name="c", subcore_axis_name="s",
                                num_cores=NC, num_subcores=NS),
    scratch_types=[
        pltpu.VMEM((SBC, CHUNK), jnp.int32),
        pltpu.VMEM((SBC, CHUNK), jnp.int32),
        pltpu.VMEM((SBC, CHUNK), jnp.int32),
        pltpu.VMEM((SBC, CHUNK), jnp.float32),
        pltpu.VMEM((CHUNK, HALF), jnp.float32),
        pltpu.VMEM((CHUNK, HALF), jnp.float32),
        pltpu.VMEM_SHARED((NPAD, HALF), jnp.float32),
        pltpu.SemaphoreType.DMA,
        pltpu.SemaphoreType.DMA,
        pltpu.SemaphoreType.DMA,
        pltpu.SemaphoreType.DMA,
    ],
)
def _sc_agg(hs_hbm, src_hbm, dst_hbm, w_hbm, zeros_hbm, out_hbm,
            srcS0, srcS1, dst_sb, w_sb, rowsA, rowsB, acc_sh,
            gsA, gsB, ssA, ssB):
    _sc_agg_body(hs_hbm, src_hbm, dst_hbm, w_hbm, zeros_hbm, out_hbm,
                 srcS0, srcS1, dst_sb, w_sb, rowsA, rowsB, acc_sh,
                 gsA, gsB, ssA, ssB)


def kernel(x, edge_index, edge_attr, batch, W1, W2, gamma1, beta1):
    src = edge_index[0]
    dst = edge_index[1]
    pad = EPAD - E
    srcp = jnp.concatenate([src, jnp.zeros((pad,), jnp.int32)])
    dstp = jnp.concatenate([dst, jnp.zeros((pad,), jnp.int32)])
    wp = jnp.concatenate([edge_attr, jnp.zeros((pad,), jnp.float32)])
    order = jnp.argsort((dstp >= 5120).astype(jnp.int32), stable=True)
    srcp = srcp[order]
    dstp = dstp[order]
    wp = wp[order]
    src5 = jnp.stack([srcp, srcp + N]).reshape(NC, NS, SB, SBC, CHUNK)
    dst5 = dstp.reshape(NS, SB, SBC, CHUNK)
    w5 = wp.reshape(NS, SB, SBC, CHUNK)
    zeros = jnp.zeros((ROWS_PER_TILE, HALF), jnp.float32)

    scale = (gamma1 * np.float32(1.0 / np.sqrt(1.0 + EPS))).reshape(2, 1, HALF)
    beta = beta1.reshape(2, 1, HALF)

    h1 = _mm_split(x, W1)
    a1 = _sc_agg(h1.reshape(2 * N, HALF), src5, dst5, w5, zeros)
    h2 = _mm_bn_split(a1.reshape(2, NPAD, HALF), W2, scale, beta)
    a2 = _sc_agg(h2.reshape(2 * N, HALF), src5, dst5, w5, zeros)
    r = a2.reshape(2, NPAD, HALF)
    return jnp.concatenate([r[0, :N], r[1, :N]], axis=1)


# speedup vs baseline: 1.3098x; 1.3098x over previous
import functools

import jax
import jax.numpy as jnp
import numpy as np
from jax import lax
from jax.experimental import pallas as pl
from jax.experimental.pallas import tpu as pltpu
from jax.experimental.pallas import tpu_sc as plsc

N = 10000
E = 160000
D = 256
H = 256
EPS = 1e-5

NC = 2
NS = 16
LANES = 16
HALF = H // 2

CHUNK = 80
EPT = 10240
EPAD = NS * EPT
NCHUNK = EPT // CHUNK
SBC = 16
SB = NCHUNK // SBC
NPAD = 10240
ROWS_PER_TILE = NPAD // NS

MM_BLK = 400


def _mm_split_body(x_ref, w_ref, o_ref):
    h = jnp.dot(x_ref[...], w_ref[...], preferred_element_type=jnp.float32)
    o_ref[0] = h[:, :HALF]
    o_ref[1] = h[:, HALF:]


def _mm_split(x, w):
    return pl.pallas_call(
        _mm_split_body,
        grid=(N // MM_BLK,),
        in_specs=[
            pl.BlockSpec((MM_BLK, D), lambda i: (i, 0)),
            pl.BlockSpec((D, H), lambda i: (0, 0)),
        ],
        out_specs=pl.BlockSpec((2, MM_BLK, HALF), lambda i: (0, i, 0)),
        out_shape=jax.ShapeDtypeStruct((2, N, HALF), jnp.float32),
    )(x, w)


def _mm_bn_body(a_ref, w_ref, s_ref, b_ref, o_ref):
    h = jnp.maximum(a_ref[...] * s_ref[...] + b_ref[...], 0.0)
    out = jnp.dot(h, w_ref[...], preferred_element_type=jnp.float32)
    o_ref[0] = out[:, :HALF]
    o_ref[1] = out[:, HALF:]


def _mm_bn_split(a, w, scale, beta):
    return pl.pallas_call(
        _mm_bn_body,
        grid=(N // MM_BLK,),
        in_specs=[
            pl.BlockSpec((MM_BLK, H), lambda i: (i, 0)),
            pl.BlockSpec((H, H), lambda i: (0, 0)),
            pl.BlockSpec((1, H), lambda i: (0, 0)),
            pl.BlockSpec((1, H), lambda i: (0, 0)),
        ],
        out_specs=pl.BlockSpec((2, MM_BLK, HALF), lambda i: (0, i, 0)),
        out_shape=jax.ShapeDtypeStruct((2, N, HALF), jnp.float32),
    )(a, w, scale, beta)


def _sc_agg_body(hs_hbm, src_hbm, dst_hbm, w_hbm, zeros_hbm, out_hbm,
                 srcS0, srcS1, dst_sb, w_sb, rowsA, rowsB, acc_sh,
                 gsA, gsB, ssA, ssB):
    c = lax.axis_index("c")
    s = lax.axis_index("s")

    pltpu.sync_copy(zeros_hbm, acc_sh.at[pl.ds(s * ROWS_PER_TILE, ROWS_PER_TILE)])
    plsc.subcore_barrier()

    def mul_rows(rows, local):
        def grp_body(g, _):
            wvec = w_sb[local, pl.ds(g * LANES, LANES)]
            for i16 in range(LANES):
                wsplat = jnp.full((LANES,), wvec[i16], dtype=jnp.float32)
                i = g * LANES + i16
                for j in range(HALF // LANES):
                    sl = pl.ds(j * LANES, LANES)
                    rows[i, sl] = rows[i, sl] * wsplat
            return ()
        lax.fori_loop(0, CHUNK // LANES, grp_body, ())

    def g_start(src_ref, local, rows, sem):
        pltpu.async_copy(hs_hbm.at[src_ref.at[local]], rows, sem)

    def g_wait(src_ref, rows, sem):
        pltpu.make_async_copy(hs_hbm.at[src_ref.at[0]], rows, sem).wait()

    def s_start(rows, local, sem):
        return pltpu.async_copy(rows, acc_sh.at[dst_sb.at[local]], sem, add=True)

    srcS = (srcS0, srcS1)
    for sb in range(SB):
        cur = srcS[sb % 2]
        nxt = srcS[(sb + 1) % 2]
        if sb == 0:
            pltpu.sync_copy(src_hbm.at[c, s, 0], cur)
        pltpu.sync_copy(dst_hbm.at[s, sb], dst_sb)
        pltpu.sync_copy(w_hbm.at[s, sb], w_sb)
        if sb + 1 < SB:
            pltpu.sync_copy(src_hbm.at[c, s, sb + 1], nxt)
        if sb == 0:
            g_start(cur, 0, rowsA, gsA)
            g_start(cur, 1, rowsB, gsB)

        def pair(m, _):
            a = 2 * m
            b = a + 1
            g_wait(cur, rowsA, gsA)
            mul_rows(rowsA, a)
            dA = s_start(rowsA, a, ssA)
            g_wait(cur, rowsB, gsB)
            mul_rows(rowsB, b)
            dB = s_start(rowsB, b, ssB)
            dA.wait()
            g_start(cur, a + 2, rowsA, gsA)
            dB.wait()
            g_start(cur, b + 2, rowsB, gsB)
            return ()

        lax.fori_loop(0, SBC // 2 - 1, pair, ())

        g_wait(cur, rowsA, gsA)
        mul_rows(rowsA, SBC - 2)
        dA = s_start(rowsA, SBC - 2, ssA)
        g_wait(cur, rowsB, gsB)
        mul_rows(rowsB, SBC - 1)
        dB = s_start(rowsB, SBC - 1, ssB)
        dA.wait()
        dB.wait()
        if sb + 1 < SB:
            g_start(nxt, 0, rowsA, gsA)
            g_start(nxt, 1, rowsB, gsB)

    plsc.subcore_barrier()

    r0 = s * ROWS_PER_TILE
    pltpu.sync_copy(acc_sh.at[pl.ds(r0, ROWS_PER_TILE)],
                    out_hbm.at[pl.ds(r0, ROWS_PER_TILE),
                               pl.ds(c * HALF, HALF)])


@functools.partial(
    pl.kernel,
    out_type=jax.ShapeDtypeStruct((NPAD, H), jnp.float32),
    mesh=plsc.VectorSubcoreMesh(core_axis_name="c", subcore_axis_name="s",
                                num_cores=NC, num_subcores=NS),
    scratch_types=[
        pltpu.VMEM((SBC, CHUNK), jnp.int32),
        pltpu.VMEM((SBC, CHUNK), jnp.int32),
        pltpu.VMEM((SBC, CHUNK), jnp.int32),
        pltpu.VMEM((SBC, CHUNK), jnp.float32),
        pltpu.VMEM((CHUNK, HALF), jnp.float32),
        pltpu.VMEM((CHUNK, HALF), jnp.float32),
        pltpu.VMEM_SHARED((NPAD, HALF), jnp.float32),
        pltpu.SemaphoreType.DMA,
        pltpu.SemaphoreType.DMA,
        pltpu.SemaphoreType.DMA,
        pltpu.SemaphoreType.DMA,
    ],
)
def _sc_agg(hs_hbm, src_hbm, dst_hbm, w_hbm, zeros_hbm, out_hbm,
            srcS0, srcS1, dst_sb, w_sb, rowsA, rowsB, acc_sh,
            gsA, gsB, ssA, ssB):
    _sc_agg_body(hs_hbm, src_hbm, dst_hbm, w_hbm, zeros_hbm, out_hbm,
                 srcS0, srcS1, dst_sb, w_sb, rowsA, rowsB, acc_sh,
                 gsA, gsB, ssA, ssB)


def kernel(x, edge_index, edge_attr, batch, W1, W2, gamma1, beta1):
    src = edge_index[0]
    dst = edge_index[1]
    pad = EPAD - E
    srcp = jnp.concatenate([src, jnp.zeros((pad,), jnp.int32)])
    dstp = jnp.concatenate([dst, jnp.zeros((pad,), jnp.int32)])
    wp = jnp.concatenate([edge_attr, jnp.zeros((pad,), jnp.float32)])
    src5 = jnp.stack([srcp, srcp + N]).reshape(NC, NS, SB, SBC, CHUNK)
    dst5 = dstp.reshape(NS, SB, SBC, CHUNK)
    w5 = wp.reshape(NS, SB, SBC, CHUNK)
    zeros = jnp.zeros((ROWS_PER_TILE, HALF), jnp.float32)

    scale = (gamma1 * np.float32(1.0 / np.sqrt(1.0 + EPS))).reshape(1, H)
    beta = beta1.reshape(1, H)

    h1 = _mm_split(x, W1)
    a1 = _sc_agg(h1.reshape(2 * N, HALF), src5, dst5, w5, zeros)
    h2 = _mm_bn_split(a1, W2, scale, beta)
    a2 = _sc_agg(h2.reshape(2 * N, HALF), src5, dst5, w5, zeros)
    return a2[:N]
